# baseline (speedup 1.0000x reference)
import dataclasses
import functools

import jax
import jax.numpy as jnp
from jax import lax
from jax.experimental import pallas as pl
from jax.experimental.pallas import tpu as pltpu
from jax.experimental.pallas import tpu_sc as plsc

N_NODES = 10000
N_EDGES = 320000
IN_DIM = 128
HID_DIM = 64
OUT_DIM = 3
D2 = 16

NC = 2
NS = 16
NW = NC * NS

R = N_NODES + 112
K = 128
NBUF = 8
NCHUNK = 80
WE = NCHUNK * K
EPAD = NW * WE

_mesh = plsc.VectorSubcoreMesh(core_axis_name="c", subcore_axis_name="s")

_cp = pltpu.CompilerParams()
if "needs_layout_passes" in pltpu.CompilerParams.__dataclass_fields__:
    _cp = dataclasses.replace(_cp, needs_layout_passes=False)
_cp = dataclasses.replace(_cp, use_tc_tiling_on_sc=False)


DEG_WE = N_EDGES // NW


@functools.partial(
    pl.kernel,
    mesh=_mesh,
    compiler_params=_cp,
    out_type=jax.ShapeDtypeStruct((NW, R), jnp.float32),
    scratch_types=[
        pltpu.VMEM((DEG_WE,), jnp.int32),
        pltpu.VMEM((R,), jnp.float32),
    ],
)
def _sc_degree(ei_hbm, hist_hbm, idx_v, hist_v):
    wid = lax.axis_index("s") * NC + lax.axis_index("c")
    zeros16 = jnp.zeros((16,), jnp.float32)
    ones16 = jnp.ones((16,), jnp.float32)

    pltpu.sync_copy(ei_hbm.at[1, pl.ds(wid * DEG_WE, DEG_WE)], idx_v)

    @pl.loop(0, R, step=16)
    def _zero(i):
        hist_v[pl.ds(i, 16)] = zeros16

    main = (DEG_WE // 128) * 128

    @pl.loop(0, main, step=128)
    def _count(i):
        for j in range(8):
            idx = idx_v[pl.ds(i + j * 16, 16)]
            plsc.addupdate_scatter(hist_v, [idx], ones16)

    @pl.loop(main, DEG_WE, step=16)
    def _count_tail(i):
        idx = idx_v[pl.ds(i, 16)]
        plsc.addupdate_scatter(hist_v, [idx], ones16)

    pltpu.sync_copy(hist_v, hist_hbm.at[wid])


def _make_sc_scatter(D):
    @functools.partial(
        pl.kernel,
        mesh=_mesh,
        compiler_params=_cp,
        out_type=jax.ShapeDtypeStruct((NC, R, D), jnp.float32),
        scratch_types=[
            pltpu.VMEM((NCHUNK, K), jnp.int32),
            pltpu.VMEM((NCHUNK, K), jnp.int32),
            pltpu.VMEM((NBUF, K, D), jnp.float32),
            pltpu.VMEM_SHARED((R, D), jnp.float32),
            pltpu.SemaphoreType.DMA((NBUF,)),
            pltpu.SemaphoreType.DMA((NBUF,)),
        ],
    )
    def _sc_scatter(table_hbm, ed_hbm, out_hbm,
                    sidx_v, didx_v, rows_v, acc_sh, gsem, ssem):
        c = lax.axis_index("c")
        s = lax.axis_index("s")
        wid = s * NC + c
        rpz = R // NS
        rslice = pl.ds(s * rpz, rpz)
        pltpu.async_copy(ed_hbm.at[0, wid], sidx_v, gsem.at[0])
        pltpu.async_copy(ed_hbm.at[1, wid], didx_v, gsem.at[1])
        zeros16 = jnp.zeros((16,), jnp.float32)

        @pl.loop(0, K)
        def _zrow(r):
            for j in range(D // 16):
                rows_v[0, r, pl.ds(j * 16, 16)] = zeros16

        nzcop = rpz // K
        for z in range(nzcop):
            pltpu.sync_copy(rows_v.at[0],
                            acc_sh.at[pl.ds(s * rpz + z * K, K)])
        rem = rpz - nzcop * K
        if rem:
            pltpu.sync_copy(rows_v.at[0].at[pl.ds(0, rem)],
                            acc_sh.at[pl.ds(s * rpz + nzcop * K, rem)])
        pltpu.make_async_copy(ed_hbm.at[0, wid], sidx_v, gsem.at[0]).wait()
        pltpu.make_async_copy(ed_hbm.at[1, wid], didx_v, gsem.at[1]).wait()
        plsc.subcore_barrier()

        def gather(ci, b):
            return pltpu.async_copy(table_hbm.at[sidx_v.at[ci]],
                                    rows_v.at[b], gsem.at[b])

        def scatter(ci, b):
            return pltpu.async_copy(rows_v.at[b], acc_sh.at[didx_v.at[ci]],
                                    ssem.at[b], add=True)

        for b in range(NBUF):
            gather(b, b)

        @pl.loop(0, NCHUNK - NBUF, step=NBUF)
        def _group(ci):
            for b in range(NBUF):
                pltpu.make_async_copy(table_hbm.at[sidx_v.at[ci + b]],
                                      rows_v.at[b], gsem.at[b]).wait()
                scatter(ci + b, b)
            for b in range(NBUF):
                pltpu.make_async_copy(rows_v.at[b],
                                      acc_sh.at[didx_v.at[ci + b]],
                                      ssem.at[b]).wait()
                gather(ci + b + NBUF, b)

        last = NCHUNK - NBUF
        for b in range(NBUF):
            pltpu.make_async_copy(table_hbm.at[sidx_v.at[last + b]],
                                  rows_v.at[b], gsem.at[b]).wait()
            scatter(last + b, b)
        for b in range(NBUF):
            pltpu.make_async_copy(rows_v.at[b],
                                  acc_sh.at[didx_v.at[last + b]],
                                  ssem.at[b]).wait()

        plsc.subcore_barrier()
        pltpu.sync_copy(acc_sh.at[rslice], out_hbm.at[c].at[rslice])

    return _sc_scatter


_sc_scatter_h = _make_sc_scatter(HID_DIM)
_sc_scatter_o = _make_sc_scatter(D2)


def _tc_scale1(xp_ref, w1_ref, hist_ref, h1_ref, d_ref):
    ones_w = jnp.ones((NW, 1), jnp.float32)
    deg = lax.dot_general(hist_ref[...], ones_w, (((0,), (0,)), ((), ())),
                          preferred_element_type=jnp.float32)
    d = lax.rsqrt(deg + 1.0)
    h1_ref[...] = jnp.dot(xp_ref[...], w1_ref[...],
                          preferred_element_type=jnp.float32)
    d_ref[...] = d


def _tc_mm2(out1_ref, w2_ref, h2_ref):
    h2_ref[...] = jnp.dot(out1_ref[...], w2_ref[...],
                          preferred_element_type=jnp.float32)


def kernel(x, edge_index, W1, b1, W2, b2):
    ei = edge_index.astype(jnp.int32)
    pad_n = EPAD - N_EDGES
    pad_idx = (jnp.arange(pad_n, dtype=jnp.int32) % (R - N_NODES)) + N_NODES
    pad2 = jnp.broadcast_to(pad_idx, (2, pad_n))
    ed = jnp.concatenate([ei, pad2], axis=1).reshape(2, NW, NCHUNK, K)

    xp = jnp.pad(x, ((0, R - N_NODES), (0, 0)))
    b1r = b1.reshape(1, HID_DIM)
    w2p = jnp.pad(W2, ((0, 0), (0, D2 - OUT_DIM)))
    b2r = jnp.pad(b2, (0, D2 - OUT_DIM)).reshape(1, D2)

    hist = _sc_degree(ei)

    h1, d_col = pl.pallas_call(
        _tc_scale1,
        out_shape=[jax.ShapeDtypeStruct((R, HID_DIM), jnp.float32),
                   jax.ShapeDtypeStruct((R, 1), jnp.float32)],
    )(xp, W1, hist)

    g1 = h1 * d_col

    acc1 = _sc_scatter_h(g1, ed)

    out1 = jnp.maximum(d_col * (acc1[0] + acc1[1] + g1) + b1r, 0.0)

    h2 = pl.pallas_call(
        _tc_mm2,
        out_shape=jax.ShapeDtypeStruct((R, D2), jnp.float32),
    )(out1, w2p)

    g2 = h2 * d_col

    acc2 = _sc_scatter_o(g2, ed)

    out = d_col * (acc2[0] + acc2[1] + g2) + b2r
    return out[:N_NODES, :OUT_DIM]


# speedup vs baseline: 1.0226x; 1.0226x over previous
import dataclasses
import functools

import jax
import jax.numpy as jnp
from jax import lax
from jax.experimental import pallas as pl
from jax.experimental.pallas import tpu as pltpu
from jax.experimental.pallas import tpu_sc as plsc

N_NODES = 10000
N_EDGES = 320000
IN_DIM = 128
HID_DIM = 64
OUT_DIM = 3
D2 = 16

NC = 2
NS = 16
NW = NC * NS

R = N_NODES + 112
K = 128
NBUF = 8
NCHUNK = 80
WE = NCHUNK * K
EPAD = NW * WE

_mesh = plsc.VectorSubcoreMesh(core_axis_name="c", subcore_axis_name="s")

_cp = pltpu.CompilerParams()
if "needs_layout_passes" in pltpu.CompilerParams.__dataclass_fields__:
    _cp = dataclasses.replace(_cp, needs_layout_passes=False)
_cp = dataclasses.replace(_cp, use_tc_tiling_on_sc=False)


DEG_WE = N_EDGES // NW


@functools.partial(
    pl.kernel,
    mesh=_mesh,
    compiler_params=_cp,
    out_type=jax.ShapeDtypeStruct((NW, R), jnp.float32),
    scratch_types=[
        pltpu.VMEM((DEG_WE,), jnp.int32),
        pltpu.VMEM((R,), jnp.float32),
    ],
)
def _sc_degree(ei_hbm, hist_hbm, idx_v, hist_v):
    wid = lax.axis_index("s") * NC + lax.axis_index("c")
    zeros16 = jnp.zeros((16,), jnp.float32)
    ones16 = jnp.ones((16,), jnp.float32)

    pltpu.sync_copy(ei_hbm.at[1, pl.ds(wid * DEG_WE, DEG_WE)], idx_v)

    @pl.loop(0, R, step=16)
    def _zero(i):
        hist_v[pl.ds(i, 16)] = zeros16

    main = (DEG_WE // 128) * 128

    @pl.loop(0, main, step=128)
    def _count(i):
        for j in range(8):
            idx = idx_v[pl.ds(i + j * 16, 16)]
            plsc.addupdate_scatter(hist_v, [idx], ones16)

    @pl.loop(main, DEG_WE, step=16)
    def _count_tail(i):
        idx = idx_v[pl.ds(i, 16)]
        plsc.addupdate_scatter(hist_v, [idx], ones16)

    pltpu.sync_copy(hist_v, hist_hbm.at[wid])


def _make_sc_scatter(D):
    @functools.partial(
        pl.kernel,
        mesh=_mesh,
        compiler_params=_cp,
        out_type=jax.ShapeDtypeStruct((NC, R, D), jnp.float32),
        scratch_types=[
            pltpu.VMEM((NCHUNK, K), jnp.int32),
            pltpu.VMEM((NCHUNK, K), jnp.int32),
            pltpu.VMEM((NBUF, K, D), jnp.float32),
            pltpu.VMEM_SHARED((R, D), jnp.float32),
            pltpu.SemaphoreType.DMA((NBUF,)),
            pltpu.SemaphoreType.DMA((NBUF,)),
        ],
    )
    def _sc_scatter(table_hbm, ed_hbm, out_hbm,
                    sidx_v, didx_v, rows_v, acc_sh, gsem, ssem):
        c = lax.axis_index("c")
        s = lax.axis_index("s")
        wid = s * NC + c
        rpz = R // NS
        rslice = pl.ds(s * rpz, rpz)
        pltpu.async_copy(ed_hbm.at[0, wid], sidx_v, gsem.at[0])
        pltpu.async_copy(ed_hbm.at[1, wid], didx_v, gsem.at[1])
        zeros16 = jnp.zeros((16,), jnp.float32)

        @pl.loop(0, K)
        def _zrow(r):
            for j in range(D // 16):
                rows_v[0, r, pl.ds(j * 16, 16)] = zeros16

        nzcop = rpz // K
        for z in range(nzcop):
            pltpu.sync_copy(rows_v.at[0],
                            acc_sh.at[pl.ds(s * rpz + z * K, K)])
        rem = rpz - nzcop * K
        if rem:
            pltpu.sync_copy(rows_v.at[0].at[pl.ds(0, rem)],
                            acc_sh.at[pl.ds(s * rpz + nzcop * K, rem)])
        pltpu.make_async_copy(ed_hbm.at[0, wid], sidx_v, gsem.at[0]).wait()
        pltpu.make_async_copy(ed_hbm.at[1, wid], didx_v, gsem.at[1]).wait()
        plsc.subcore_barrier()

        def gather(ci, b):
            return pltpu.async_copy(table_hbm.at[sidx_v.at[ci]],
                                    rows_v.at[b], gsem.at[b])

        def scatter(ci, b):
            return pltpu.async_copy(rows_v.at[b], acc_sh.at[didx_v.at[ci]],
                                    ssem.at[b], add=True)

        for b in range(NBUF):
            gather(b, b)

        @pl.loop(0, NCHUNK - NBUF, step=NBUF)
        def _group(ci):
            for b in range(NBUF):
                pltpu.make_async_copy(table_hbm.at[sidx_v.at[ci + b]],
                                      rows_v.at[b], gsem.at[b]).wait()
                scatter(ci + b, b)
            for b in range(NBUF):
                pltpu.make_async_copy(rows_v.at[b],
                                      acc_sh.at[didx_v.at[ci + b]],
                                      ssem.at[b]).wait()
                gather(ci + b + NBUF, b)

        last = NCHUNK - NBUF
        for b in range(NBUF):
            pltpu.make_async_copy(table_hbm.at[sidx_v.at[last + b]],
                                  rows_v.at[b], gsem.at[b]).wait()
            scatter(last + b, b)
        for b in range(NBUF):
            pltpu.make_async_copy(rows_v.at[b],
                                  acc_sh.at[didx_v.at[last + b]],
                                  ssem.at[b]).wait()

        plsc.subcore_barrier()
        pltpu.sync_copy(acc_sh.at[rslice], out_hbm.at[c].at[rslice])

    return _sc_scatter


_sc_scatter_h = _make_sc_scatter(HID_DIM)
_sc_scatter_o = _make_sc_scatter(D2)


def _tc_scale1(xp_ref, w1_ref, hist_ref, g1_ref, d_ref):
    ones_w = jnp.ones((NW, 1), jnp.float32)
    deg = lax.dot_general(hist_ref[...], ones_w, (((0,), (0,)), ((), ())),
                          preferred_element_type=jnp.float32)
    d = lax.rsqrt(deg + 1.0)
    h1 = jnp.dot(xp_ref[...], w1_ref[...], preferred_element_type=jnp.float32)
    g1_ref[...] = h1 * d
    d_ref[...] = d


def _tc_mid(d_ref, acc1_ref, g1_ref, b1_ref, w2_ref, g2_ref):
    d = d_ref[...]
    out1 = jnp.maximum(d * (acc1_ref[...] + g1_ref[...]) + b1_ref[...], 0.0)
    h2 = jnp.dot(out1, w2_ref[...], preferred_element_type=jnp.float32)
    g2_ref[...] = h2 * d


def _tc_final(d_ref, acc2_ref, g2_ref, b2_ref, out_ref):
    d = d_ref[...]
    full = d * (acc2_ref[...] + g2_ref[...]) + b2_ref[...]
    out_ref[...] = full[:N_NODES, :OUT_DIM]


def kernel(x, edge_index, W1, b1, W2, b2):
    ei = edge_index.astype(jnp.int32)
    pad_n = EPAD - N_EDGES
    pad_idx = (jnp.arange(pad_n, dtype=jnp.int32) % (R - N_NODES)) + N_NODES
    pad2 = jnp.broadcast_to(pad_idx, (2, pad_n))
    ed = jnp.concatenate([ei, pad2], axis=1).reshape(2, NW, NCHUNK, K)

    xp = jnp.pad(x, ((0, R - N_NODES), (0, 0)))
    b1r = b1.reshape(1, HID_DIM)
    w2p = jnp.pad(W2, ((0, 0), (0, D2 - OUT_DIM)))
    b2r = jnp.pad(b2, (0, D2 - OUT_DIM)).reshape(1, D2)

    hist = _sc_degree(ei)

    g1, d_col = pl.pallas_call(
        _tc_scale1,
        out_shape=[jax.ShapeDtypeStruct((R, HID_DIM), jnp.float32),
                   jax.ShapeDtypeStruct((R, 1), jnp.float32)],
    )(xp, W1, hist)

    acc1 = _sc_scatter_h(g1, ed)
    acc1s = acc1[0] + acc1[1]

    g2 = pl.pallas_call(
        _tc_mid,
        out_shape=jax.ShapeDtypeStruct((R, D2), jnp.float32),
    )(d_col, acc1s, g1, b1r, w2p)

    acc2 = _sc_scatter_o(g2, ed)
    acc2s = acc2[0] + acc2[1]

    out = pl.pallas_call(
        _tc_final,
        out_shape=jax.ShapeDtypeStruct((N_NODES, OUT_DIM), jnp.float32),
    )(d_col, acc2s, g2, b2r)

    return out


# speedup vs baseline: 1.1019x; 1.0775x over previous
import dataclasses
import functools

import jax
import jax.numpy as jnp
from jax import lax
from jax.experimental import pallas as pl
from jax.experimental.pallas import tpu as pltpu
from jax.experimental.pallas import tpu_sc as plsc

N_NODES = 10000
N_EDGES = 320000
IN_DIM = 128
HID_DIM = 64
OUT_DIM = 3
D2 = 16

NC = 2
NS = 16
NW = NC * NS

R = N_NODES + 112
K = 128
NBUF = 8
NCHUNK = 80
WE = NCHUNK * K
EPAD = NW * WE

_mesh = plsc.VectorSubcoreMesh(core_axis_name="c", subcore_axis_name="s")

_cp = pltpu.CompilerParams()
if "needs_layout_passes" in pltpu.CompilerParams.__dataclass_fields__:
    _cp = dataclasses.replace(_cp, needs_layout_passes=False)
_cp = dataclasses.replace(_cp, use_tc_tiling_on_sc=False)


DEG_WE = N_EDGES // NW


@functools.partial(
    pl.kernel,
    mesh=_mesh,
    compiler_params=_cp,
    out_type=jax.ShapeDtypeStruct((NW, R), jnp.float32),
    scratch_types=[
        pltpu.VMEM((DEG_WE,), jnp.int32),
        pltpu.VMEM((R,), jnp.float32),
    ],
)
def _sc_degree(ei_hbm, hist_hbm, idx_v, hist_v):
    wid = lax.axis_index("s") * NC + lax.axis_index("c")
    zeros16 = jnp.zeros((16,), jnp.float32)
    ones16 = jnp.ones((16,), jnp.float32)

    pltpu.sync_copy(ei_hbm.at[1, pl.ds(wid * DEG_WE, DEG_WE)], idx_v)

    @pl.loop(0, R, step=16)
    def _zero(i):
        hist_v[pl.ds(i, 16)] = zeros16

    main = (DEG_WE // 128) * 128

    @pl.loop(0, main, step=128)
    def _count(i):
        for j in range(8):
            idx = idx_v[pl.ds(i + j * 16, 16)]
            plsc.addupdate_scatter(hist_v, [idx], ones16)

    @pl.loop(main, DEG_WE, step=16)
    def _count_tail(i):
        idx = idx_v[pl.ds(i, 16)]
        plsc.addupdate_scatter(hist_v, [idx], ones16)

    pltpu.sync_copy(hist_v, hist_hbm.at[wid])


def _make_sc_scatter(D):
    @functools.partial(
        pl.kernel,
        mesh=_mesh,
        compiler_params=_cp,
        out_type=jax.ShapeDtypeStruct((NC, R, D), jnp.float32),
        scratch_types=[
            pltpu.VMEM((NCHUNK, K), jnp.int32),
            pltpu.VMEM((NCHUNK, K), jnp.int32),
            pltpu.VMEM((NBUF, K, D), jnp.float32),
            pltpu.VMEM_SHARED((R, D), jnp.float32),
            pltpu.SemaphoreType.DMA((NBUF,)),
            pltpu.SemaphoreType.DMA((NBUF,)),
        ],
    )
    def _sc_scatter(table_hbm, ed_hbm, out_hbm,
                    sidx_v, didx_v, rows_v, acc_sh, gsem, ssem):
        c = lax.axis_index("c")
        s = lax.axis_index("s")
        wid = s * NC + c
        rpz = R // NS
        rslice = pl.ds(s * rpz, rpz)
        pltpu.async_copy(ed_hbm.at[0, wid], sidx_v, gsem.at[0])
        pltpu.async_copy(ed_hbm.at[1, wid], didx_v, gsem.at[1])
        zeros16 = jnp.zeros((16,), jnp.float32)

        @pl.loop(0, K)
        def _zrow(r):
            for j in range(D // 16):
                rows_v[0, r, pl.ds(j * 16, 16)] = zeros16

        nzcop = rpz // K
        for z in range(nzcop):
            pltpu.sync_copy(rows_v.at[0],
                            acc_sh.at[pl.ds(s * rpz + z * K, K)])
        rem = rpz - nzcop * K
        if rem:
            pltpu.sync_copy(rows_v.at[0].at[pl.ds(0, rem)],
                            acc_sh.at[pl.ds(s * rpz + nzcop * K, rem)])
        pltpu.make_async_copy(ed_hbm.at[0, wid], sidx_v, gsem.at[0]).wait()
        pltpu.make_async_copy(ed_hbm.at[1, wid], didx_v, gsem.at[1]).wait()
        plsc.subcore_barrier()

        def gather(ci, b):
            return pltpu.async_copy(table_hbm.at[sidx_v.at[ci]],
                                    rows_v.at[b], gsem.at[b])

        def scatter(ci, b):
            return pltpu.async_copy(rows_v.at[b], acc_sh.at[didx_v.at[ci]],
                                    ssem.at[b], add=True)

        for b in range(NBUF):
            gather(b, b)

        @pl.loop(0, NCHUNK - NBUF, step=NBUF)
        def _group(ci):
            for b in range(NBUF):
                pltpu.make_async_copy(table_hbm.at[sidx_v.at[ci + b]],
                                      rows_v.at[b], gsem.at[b]).wait()
                scatter(ci + b, b)
            for b in range(NBUF):
                pltpu.make_async_copy(rows_v.at[b],
                                      acc_sh.at[didx_v.at[ci + b]],
                                      ssem.at[b]).wait()
                gather(ci + b + NBUF, b)

        last = NCHUNK - NBUF
        for b in range(NBUF):
            pltpu.make_async_copy(table_hbm.at[sidx_v.at[last + b]],
                                  rows_v.at[b], gsem.at[b]).wait()
            scatter(last + b, b)
        for b in range(NBUF):
            pltpu.make_async_copy(rows_v.at[b],
                                  acc_sh.at[didx_v.at[last + b]],
                                  ssem.at[b]).wait()

        plsc.subcore_barrier()
        pltpu.sync_copy(acc_sh.at[rslice], out_hbm.at[c].at[rslice])

    return _sc_scatter


_sc_scatter_h = _make_sc_scatter(HID_DIM)
_sc_scatter_o = _make_sc_scatter(D2)


def _tc_scale1(xp_ref, w1_ref, hist_ref, g1_ref, d_ref):
    ones_w = jnp.ones((NW, 1), jnp.float32)
    deg = lax.dot_general(hist_ref[...], ones_w, (((0,), (0,)), ((), ())),
                          preferred_element_type=jnp.float32)
    d = lax.rsqrt(deg + 1.0)
    h1 = jnp.dot(xp_ref[...], w1_ref[...], preferred_element_type=jnp.float32)
    g1_ref[:N_NODES, :] = h1 * d[:N_NODES]
    g1_ref[N_NODES:, :] = jnp.zeros((R - N_NODES, HID_DIM), jnp.float32)
    d_ref[...] = d


def _tc_mid(d_ref, acc1_ref, g1_ref, b1_ref, w2_ref, g2_ref):
    d = d_ref[...]
    acc1 = acc1_ref[0] + acc1_ref[1]
    out1 = jnp.maximum(d * (acc1 + g1_ref[...]) + b1_ref[...], 0.0)
    h2 = jnp.dot(out1, w2_ref[...], preferred_element_type=jnp.float32)
    g2_ref[...] = h2 * d


def _tc_final(d_ref, acc2_ref, g2_ref, b2_ref, out_ref):
    d = d_ref[...]
    acc2 = acc2_ref[0] + acc2_ref[1]
    full = d * (acc2 + g2_ref[...]) + b2_ref[...]
    out_ref[...] = full[:N_NODES, :OUT_DIM]


def kernel(x, edge_index, W1, b1, W2, b2):
    ei = edge_index.astype(jnp.int32)
    pad_n = EPAD - N_EDGES
    pad_idx = (jnp.arange(pad_n, dtype=jnp.int32) % (R - N_NODES)) + N_NODES
    pad2 = jnp.broadcast_to(pad_idx, (2, pad_n))
    ed = jnp.concatenate([ei, pad2], axis=1).reshape(2, NW, NCHUNK, K)

    b1r = b1.reshape(1, HID_DIM)
    w2p = jnp.pad(W2, ((0, 0), (0, D2 - OUT_DIM)))
    b2r = jnp.pad(b2, (0, D2 - OUT_DIM)).reshape(1, D2)

    hist = _sc_degree(ei)

    g1, d_col = pl.pallas_call(
        _tc_scale1,
        out_shape=[jax.ShapeDtypeStruct((R, HID_DIM), jnp.float32),
                   jax.ShapeDtypeStruct((R, 1), jnp.float32)],
    )(x, W1, hist)

    acc1 = _sc_scatter_h(g1, ed)

    g2 = pl.pallas_call(
        _tc_mid,
        out_shape=jax.ShapeDtypeStruct((R, D2), jnp.float32),
    )(d_col, acc1, g1, b1r, w2p)

    acc2 = _sc_scatter_o(g2, ed)

    out = pl.pallas_call(
        _tc_final,
        out_shape=jax.ShapeDtypeStruct((N_NODES, OUT_DIM), jnp.float32),
    )(d_col, acc2, g2, b2r)

    return out
